# baseline (speedup 1.0000x reference)
import functools

import jax
import jax.numpy as jnp
from jax import lax
from jax.experimental import pallas as pl
from jax.experimental.pallas import tpu as pltpu
from jax.experimental.pallas import tpu_sc as plsc

_VOCAB = 1000
_VPAD = 1024
_B = 1024
_T = 50
_N = _B * _T
_NC = 2
_NS = 16
_NW = _NC * _NS
_NPART = 4
_PB = _B // _NPART
_BPW = _PB // _NW
_TPW = _BPW * _T
_OFFS = (0, 16, 32, 34)


def _row_lse_body(t_ref, o_ref):
    t = t_ref[...]
    m = jnp.max(t, axis=1)
    s = jnp.sum(jnp.exp(t - m[:, None]), axis=1)
    o_ref[pl.ds(0, _VOCAB)] = jnp.log(s) + m


def _row_lse(table):
    return pl.pallas_call(
        _row_lse_body,
        out_shape=jax.ShapeDtypeStruct((_VPAD,), jnp.float32),
    )(table)


def _loss_body(p_ref, o_ref):
    o_ref[...] = (jnp.sum(p_ref[...]) * (1.0 / _N)).reshape(1, 1)


def _loss_reduce(partials):
    return pl.pallas_call(
        _loss_body,
        out_shape=jax.ShapeDtypeStruct((1, 1), jnp.float32),
    )(partials)


@functools.lru_cache(maxsize=1)
def _sc_gather_fn():
    @functools.partial(
        pl.kernel,
        out_type=(
            jax.ShapeDtypeStruct((_PB, _T, _VOCAB), jnp.float32),
            jax.ShapeDtypeStruct((_NW, 16), jnp.float32),
        ),
        mesh=plsc.VectorSubcoreMesh(core_axis_name="c", subcore_axis_name="s"),
        compiler_params=pltpu.CompilerParams(needs_layout_passes=False),
        scratch_types=[
            pltpu.VMEM((_TPW,), jnp.int32),
            pltpu.VMEM((_TPW,), jnp.int32),
            [pltpu.VMEM((8, _T), jnp.int32)] * 2,
            [pltpu.VMEM((_T,), jnp.int32)] * 2,
            [pltpu.VMEM((_T,), jnp.float32)] * 2,
            pltpu.VMEM((_T, 128), jnp.float32),
            [pltpu.VMEM((_T, _VOCAB), jnp.float32)] * 2,
            pltpu.VMEM((_VPAD,), jnp.float32),
            pltpu.VMEM((16,), jnp.float32),
            [pltpu.SemaphoreType.DMA] * 2,
            pltpu.SemaphoreType.DMA,
            [pltpu.SemaphoreType.DMA] * 2,
        ],
    )
    def _sc_gather(idx_hbm, tgt_hbm, tab8_hbm, tabflat_hbm, lse_hbm,
                   out_hbm, part_hbm,
                   idx_v, tgt_v, gidx_v, flat_v, picked_v, tail_v, stage_v,
                   lse_v, acc_v, sem_g, sem_t, sem_w):
        wid = lax.axis_index("s") * _NC + lax.axis_index("c")
        tbase = wid * _TPW
        bbase = wid * _BPW
        pltpu.sync_copy(lse_hbm, lse_v)
        pltpu.sync_copy(idx_hbm.at[pl.ds(tbase, _TPW)], idx_v)
        pltpu.sync_copy(tgt_hbm.at[pl.ds(tbase, _TPW)], tgt_v)
        acc_v[...] = jnp.zeros((16,), jnp.float32)
        lane = lax.iota(jnp.int32, 16)

        def build_indices(p, j):
            t0 = j * _T
            for o in _OFFS:
                i16 = idx_v[pl.ds(t0 + o, 16)]
                t16 = tgt_v[pl.ds(t0 + o, 16)]
                s16 = i16 * 8
                flat_v[p][pl.ds(o, 16)] = i16 * _VOCAB + t16
                for ct in range(8):
                    gidx_v[p][ct, pl.ds(o, 16)] = s16 + ct

        def fire_main(p, j):
            cps = [pltpu.async_copy(
                tab8_hbm.at[gidx_v[p].at[ct]],
                stage_v[p].at[:, pl.ds(ct * 128, 128)], sem_g[p])
                for ct in range(7)]
            cps.append(pltpu.async_copy(
                tabflat_hbm.at[flat_v[p]], picked_v[p], sem_g[p]))
            return cps

        def fire_tail(p):
            return pltpu.async_copy(tab8_hbm.at[gidx_v[p].at[7]], tail_v,
                                    sem_t)

        def tail_copy(p):
            def tail(r, c):
                for k in (0, 16, 32, 48, 64, 80, 88):
                    stage_v[p][r, pl.ds(896 + k, 16)] = tail_v[r, pl.ds(k, 16)]
                return c
            lax.fori_loop(0, _T, tail, 0, unroll=5)

        def loss_accum(p, j):
            t0 = j * _T
            for o in _OFFS:
                i16 = idx_v[pl.ds(t0 + o, 16)]
                lse16 = plsc.load_gather(lse_v, [i16])
                p16 = picked_v[p][pl.ds(o, 16)]
                contrib = lse16 - p16
                if o == 34:
                    contrib = jnp.where(lane >= 14, contrib,
                                        jnp.zeros((16,), jnp.float32))
                acc_v[...] = acc_v[...] + contrib

        def drain_write(p, g):
            @pl.when(g > 0)
            def _():
                pltpu.make_async_copy(stage_v[p], out_hbm.at[bbase],
                                      sem_w[p]).wait()

        def body(g, carry):
            j0 = g * 2
            drain_write(0, g)
            build_indices(0, j0)
            m0 = fire_main(0, j0)
            t0 = fire_tail(0)
            drain_write(1, g)
            build_indices(1, j0 + 1)
            m1 = fire_main(1, j0 + 1)
            for c in m0:
                c.wait()
            t0.wait()
            tail_copy(0)
            loss_accum(0, j0)
            pltpu.async_copy(stage_v[0], out_hbm.at[bbase + j0], sem_w[0])
            t1 = fire_tail(1)
            for c in m1:
                c.wait()
            t1.wait()
            tail_copy(1)
            loss_accum(1, j0 + 1)
            pltpu.async_copy(stage_v[1], out_hbm.at[bbase + j0 + 1], sem_w[1])
            return carry

        lax.fori_loop(0, _BPW // 2, body, 0)
        for p in (0, 1):
            pltpu.make_async_copy(stage_v[p], out_hbm.at[bbase],
                                  sem_w[p]).wait()
        pltpu.sync_copy(acc_v, part_hbm.at[wid])

    return _sc_gather


def kernel(idx, targets, table):
    table8 = jnp.pad(table, ((0, 0), (0, _VPAD - _VOCAB))).reshape(
        _VOCAB * 8, 128)
    table_flat = table.reshape(_VOCAB * _VOCAB)
    row_lse = _row_lse(table)
    sc = _sc_gather_fn()
    outs, parts = [], []
    for p in range(_NPART):
        idx_p = idx[p * _PB:(p + 1) * _PB].reshape(_PB * _T)
        tgt_p = targets[p * _PB:(p + 1) * _PB].reshape(_PB * _T)
        o, q = sc(idx_p, tgt_p, table8, table_flat, row_lse)
        outs.append(o)
        parts.append(q)
    logits = jnp.zeros((_B, _T, _VOCAB), jnp.float32)
    for p in range(_NPART):
        logits = lax.dynamic_update_slice(logits, outs[p], (p * _PB, 0, 0))
    loss = _loss_reduce(jnp.concatenate(parts, axis=0))[0, 0]
    return (logits, loss)


# speedup vs baseline: 1.0289x; 1.0289x over previous
import functools

import jax
import jax.numpy as jnp
from jax import lax
from jax.experimental import pallas as pl
from jax.experimental.pallas import tpu as pltpu
from jax.experimental.pallas import tpu_sc as plsc

_VOCAB = 1000
_VPAD = 1024
_B = 1024
_T = 50
_N = _B * _T
_NC = 2
_NS = 16
_NW = _NC * _NS
_NPART = 4
_PB = _B // _NPART
_BPW = _PB // _NW
_TPW = _BPW * _T
_OFFS = (0, 16, 32, 34)


def _row_lse_body(t_ref, o_ref):
    t = t_ref[...]
    m = jnp.max(t, axis=1)
    s = jnp.sum(jnp.exp(t - m[:, None]), axis=1)
    o_ref[pl.ds(0, _VOCAB)] = jnp.log(s) + m


def _row_lse(table):
    return pl.pallas_call(
        _row_lse_body,
        out_shape=jax.ShapeDtypeStruct((_VPAD,), jnp.float32),
    )(table)


def _loss_body(p_ref, o_ref):
    o_ref[...] = (jnp.sum(p_ref[...]) * (1.0 / _N)).reshape(1, 1)


def _loss_reduce(partials):
    return pl.pallas_call(
        _loss_body,
        out_shape=jax.ShapeDtypeStruct((1, 1), jnp.float32),
    )(partials)


@functools.lru_cache(maxsize=1)
def _sc_gather_fn():
    @functools.partial(
        pl.kernel,
        out_type=(
            jax.ShapeDtypeStruct((_PB, _T, _VOCAB), jnp.float32),
            jax.ShapeDtypeStruct((_NW, 16), jnp.float32),
        ),
        mesh=plsc.VectorSubcoreMesh(core_axis_name="c", subcore_axis_name="s"),
        compiler_params=pltpu.CompilerParams(needs_layout_passes=False),
        scratch_types=[
            pltpu.VMEM((_TPW,), jnp.int32),
            pltpu.VMEM((_TPW,), jnp.int32),
            [pltpu.VMEM((8, _T), jnp.int32)] * 2,
            [pltpu.VMEM((_T,), jnp.int32)] * 2,
            [pltpu.VMEM((_T,), jnp.float32)] * 2,
            pltpu.VMEM((_T, 128), jnp.float32),
            [pltpu.VMEM((_T, _VOCAB), jnp.float32)] * 2,
            pltpu.VMEM((_VPAD,), jnp.float32),
            pltpu.VMEM((16,), jnp.float32),
            [pltpu.SemaphoreType.DMA] * 2,
            pltpu.SemaphoreType.DMA,
            [pltpu.SemaphoreType.DMA] * 2,
        ],
    )
    def _sc_gather(idx_hbm, tgt_hbm, tab8_hbm, tabflat_hbm, lse_hbm,
                   out_hbm, part_hbm,
                   idx_v, tgt_v, gidx_v, flat_v, picked_v, tail_v, stage_v,
                   lse_v, acc_v, sem_g, sem_t, sem_w):
        wid = lax.axis_index("s") * _NC + lax.axis_index("c")
        tbase = wid * _TPW
        bbase = wid * _BPW
        pltpu.sync_copy(lse_hbm, lse_v)
        pltpu.sync_copy(idx_hbm.at[pl.ds(tbase, _TPW)], idx_v)
        pltpu.sync_copy(tgt_hbm.at[pl.ds(tbase, _TPW)], tgt_v)
        acc_v[...] = jnp.zeros((16,), jnp.float32)
        lane = lax.iota(jnp.int32, 16)

        def build_indices(p, j):
            t0 = j * _T
            for o in _OFFS:
                i16 = idx_v[pl.ds(t0 + o, 16)]
                t16 = tgt_v[pl.ds(t0 + o, 16)]
                s16 = i16 * 8
                flat_v[p][pl.ds(o, 16)] = i16 * _VOCAB + t16
                for ct in range(8):
                    gidx_v[p][ct, pl.ds(o, 16)] = s16 + ct

        def fire_main(p, j):
            cps = [pltpu.async_copy(
                tab8_hbm.at[gidx_v[p].at[ct]],
                stage_v[p].at[:, pl.ds(ct * 128, 128)], sem_g[p])
                for ct in range(7)]
            cps.append(pltpu.async_copy(
                tabflat_hbm.at[flat_v[p]], picked_v[p], sem_g[p]))
            return cps

        def fire_tail(p):
            return pltpu.async_copy(tab8_hbm.at[gidx_v[p].at[7]], tail_v,
                                    sem_t)

        def tail_copy(p):
            def tail(r, c):
                for k in (0, 16, 32, 48, 64, 80, 88):
                    stage_v[p][r, pl.ds(896 + k, 16)] = tail_v[r, pl.ds(k, 16)]
                return c
            lax.fori_loop(0, _T, tail, 0, unroll=5)

        def loss_accum(p, j):
            t0 = j * _T
            for o in _OFFS:
                i16 = idx_v[pl.ds(t0 + o, 16)]
                lse16 = plsc.load_gather(lse_v, [i16])
                p16 = picked_v[p][pl.ds(o, 16)]
                contrib = lse16 - p16
                if o == 34:
                    contrib = jnp.where(lane >= 14, contrib,
                                        jnp.zeros((16,), jnp.float32))
                acc_v[...] = acc_v[...] + contrib

        def drain_write(p, g):
            @pl.when(g > 0)
            def _():
                pltpu.make_async_copy(stage_v[p], out_hbm.at[bbase],
                                      sem_w[p]).wait()

        def body(g, carry):
            j0 = g * 2
            drain_write(0, g)
            build_indices(0, j0)
            m0 = fire_main(0, j0)
            t0 = fire_tail(0)
            drain_write(1, g)
            build_indices(1, j0 + 1)
            m1 = fire_main(1, j0 + 1)
            for c in m0:
                c.wait()
            t0.wait()
            tail_copy(0)
            loss_accum(0, j0)
            pltpu.async_copy(stage_v[0], out_hbm.at[bbase + j0], sem_w[0])
            t1 = fire_tail(1)
            for c in m1:
                c.wait()
            t1.wait()
            tail_copy(1)
            loss_accum(1, j0 + 1)
            pltpu.async_copy(stage_v[1], out_hbm.at[bbase + j0 + 1], sem_w[1])
            return carry

        lax.fori_loop(0, _BPW // 2, body, 0)
        for p in (0, 1):
            pltpu.make_async_copy(stage_v[p], out_hbm.at[bbase],
                                  sem_w[p]).wait()
        pltpu.sync_copy(acc_v, part_hbm.at[wid])

    return _sc_gather


def kernel(idx, targets, table):
    table8 = jnp.pad(table, ((0, 0), (0, _VPAD - _VOCAB))).reshape(
        _VOCAB * 8, 128)
    table_flat = table.reshape(_VOCAB * _VOCAB)
    row_lse = _row_lse(table)
    sc = _sc_gather_fn()
    outs, parts = [], []
    for p in range(_NPART):
        idx_p = idx[p * _PB:(p + 1) * _PB].reshape(_PB * _T)
        tgt_p = targets[p * _PB:(p + 1) * _PB].reshape(_PB * _T)
        o, q = sc(idx_p, tgt_p, table8, table_flat, row_lse)
        outs.append(o)
        parts.append(q)
    logits_t = jnp.concatenate(
        [o.transpose(1, 2, 0) for o in outs], axis=2)
    logits_t = lax.optimization_barrier(logits_t)
    logits = logits_t.transpose(2, 0, 1)
    loss = _loss_reduce(jnp.concatenate(parts, axis=0))[0, 0]
    return (logits, loss)


# speedup vs baseline: 1.3412x; 1.3035x over previous
import functools

import jax
import jax.numpy as jnp
from jax import lax
from jax.experimental import pallas as pl
from jax.experimental.pallas import tpu as pltpu
from jax.experimental.pallas import tpu_sc as plsc

_VOCAB = 1000
_VPAD = 1024
_B = 1024
_T = 50
_N = _B * _T
_NC = 2
_NS = 16
_NW = _NC * _NS
_NPART = 1
_PB = _B // _NPART
_BPW = _PB // _NW
_TPW = _BPW * _T
_OFFS = (0, 16, 32, 34)


def _row_lse_body(t_ref, o_ref):
    t = t_ref[...]
    m = jnp.max(t, axis=1)
    s = jnp.sum(jnp.exp(t - m[:, None]), axis=1)
    o_ref[pl.ds(0, _VOCAB)] = jnp.log(s) + m


def _row_lse(table):
    return pl.pallas_call(
        _row_lse_body,
        out_shape=jax.ShapeDtypeStruct((_VPAD,), jnp.float32),
    )(table)


def _loss_body(p_ref, o_ref):
    o_ref[...] = (jnp.sum(p_ref[...]) * (1.0 / _N)).reshape(1, 1)


def _loss_reduce(partials):
    return pl.pallas_call(
        _loss_body,
        out_shape=jax.ShapeDtypeStruct((1, 1), jnp.float32),
    )(partials)


@functools.lru_cache(maxsize=1)
def _sc_gather_fn():
    @functools.partial(
        pl.kernel,
        out_type=(
            jax.ShapeDtypeStruct((_PB, _T, _VOCAB), jnp.float32),
            jax.ShapeDtypeStruct((_NW, 16), jnp.float32),
        ),
        mesh=plsc.VectorSubcoreMesh(core_axis_name="c", subcore_axis_name="s"),
        compiler_params=pltpu.CompilerParams(needs_layout_passes=False),
        scratch_types=[
            pltpu.VMEM((_TPW,), jnp.int32),
            pltpu.VMEM((_TPW,), jnp.int32),
            [pltpu.VMEM((_T,), jnp.int32)] * 2,
            [pltpu.VMEM((_T,), jnp.int32)] * 2,
            [pltpu.VMEM((_T,), jnp.float32)] * 2,
            pltpu.VMEM((_T, 128), jnp.float32),
            [pltpu.VMEM((_T, _VOCAB), jnp.float32)] * 2,
            pltpu.VMEM((_VPAD,), jnp.float32),
            pltpu.VMEM((16,), jnp.float32),
            [pltpu.SemaphoreType.DMA] * 2,
            pltpu.SemaphoreType.DMA,
            [pltpu.SemaphoreType.DMA] * 2,
        ],
    )
    def _sc_gather(idx_hbm, tgt_hbm, tabpre_hbm, tabtail_hbm, tabflat_hbm,
                   lse_hbm, out_hbm, part_hbm,
                   idx_v, tgt_v, gpre_v, flat_v, picked_v, tail_v,
                   stage_v, lse_v, acc_v, sem_g, sem_t, sem_w):
        wid = lax.axis_index("s") * _NC + lax.axis_index("c")
        tbase = wid * _TPW
        bbase = wid * _BPW
        pltpu.sync_copy(lse_hbm, lse_v)
        pltpu.sync_copy(idx_hbm.at[pl.ds(tbase, _TPW)], idx_v)
        pltpu.sync_copy(tgt_hbm.at[pl.ds(tbase, _TPW)], tgt_v)
        acc_v[...] = jnp.zeros((16,), jnp.float32)
        lane = lax.iota(jnp.int32, 16)

        def build_indices(p, j):
            t0 = j * _T
            for o in _OFFS:
                i16 = idx_v[pl.ds(t0 + o, 16)]
                t16 = tgt_v[pl.ds(t0 + o, 16)]
                flat_v[p][pl.ds(o, 16)] = i16 * _VOCAB + t16
                gpre_v[p][pl.ds(o, 16)] = i16

        def fire_main(p, j):
            cps = [pltpu.async_copy(
                tabpre_hbm.at[gpre_v[p]],
                stage_v[p].at[:, pl.ds(0, 896)], sem_g[p])]
            cps.append(pltpu.async_copy(
                tabflat_hbm.at[flat_v[p]], picked_v[p], sem_g[p]))
            return cps

        def fire_tail(p):
            return pltpu.async_copy(tabtail_hbm.at[gpre_v[p]], tail_v,
                                    sem_t)

        def tail_copy(p):
            def tail(r, c):
                for k in (0, 16, 32, 48, 64, 80, 88):
                    stage_v[p][r, pl.ds(896 + k, 16)] = tail_v[r, pl.ds(k, 16)]
                return c
            lax.fori_loop(0, _T, tail, 0, unroll=5)

        def loss_accum(p, j):
            t0 = j * _T
            for o in _OFFS:
                i16 = idx_v[pl.ds(t0 + o, 16)]
                lse16 = plsc.load_gather(lse_v, [i16])
                p16 = picked_v[p][pl.ds(o, 16)]
                contrib = lse16 - p16
                if o == 34:
                    contrib = jnp.where(lane >= 14, contrib,
                                        jnp.zeros((16,), jnp.float32))
                acc_v[...] = acc_v[...] + contrib

        def drain_write(p, g):
            @pl.when(g > 0)
            def _():
                pltpu.make_async_copy(stage_v[p], out_hbm.at[bbase],
                                      sem_w[p]).wait()

        def body(g, carry):
            j0 = g * 2
            drain_write(0, g)
            build_indices(0, j0)
            m0 = fire_main(0, j0)
            t0 = fire_tail(0)
            drain_write(1, g)
            build_indices(1, j0 + 1)
            m1 = fire_main(1, j0 + 1)
            for c in m0:
                c.wait()
            t0.wait()
            tail_copy(0)
            loss_accum(0, j0)
            pltpu.async_copy(stage_v[0], out_hbm.at[bbase + j0], sem_w[0])
            t1 = fire_tail(1)
            for c in m1:
                c.wait()
            t1.wait()
            tail_copy(1)
            loss_accum(1, j0 + 1)
            pltpu.async_copy(stage_v[1], out_hbm.at[bbase + j0 + 1], sem_w[1])
            return carry

        lax.fori_loop(0, _BPW // 2, body, 0)
        for p in (0, 1):
            pltpu.make_async_copy(stage_v[p], out_hbm.at[bbase],
                                  sem_w[p]).wait()
        pltpu.sync_copy(acc_v, part_hbm.at[wid])

    return _sc_gather


def kernel(idx, targets, table):
    table_pre = table[:, :896]
    table_tail = jnp.pad(table[:, 896:], ((0, 0), (0, 24)))
    table_flat = table.reshape(_VOCAB * _VOCAB)
    row_lse = _row_lse(table)
    idx_f = idx.reshape(_N)
    tgt_f = targets.reshape(_N)
    logits, partials = _sc_gather_fn()(
        idx_f, tgt_f, table_pre, table_tail, table_flat, row_lse)
    loss = _loss_reduce(partials)[0, 0]
    return (logits, loss)
